# baseline (speedup 1.0000x reference)
import jax
import jax.numpy as jnp
from jax import lax
from jax.experimental import pallas as pl
from jax.experimental.pallas import tpu as pltpu
from jax.experimental.pallas import tpu_sc as plsc

NC = 2
NS = 16
NW = NC * NS
L = 16


def _sc_body(ids_hbm, table_hbm, out_hbm, idx_v, table_v, sem):
    wid = lax.axis_index("s") * NC + lax.axis_index("c")
    n_tok = ids_hbm.shape[0]
    b_per_w = n_tok // NW
    base = wid * b_per_w

    pltpu.sync_copy(table_hbm, table_v)
    pltpu.sync_copy(ids_hbm.at[pl.ds(base, b_per_w)], idx_v)

    UNROLL = 4

    def tok_body(i, carry):
        ts = [i * UNROLL + u for u in range(UNROLL)]
        id_vecs = [
            plsc.load_gather(idx_v, [jnp.full((L,), t, jnp.int32)])
            for t in ts
        ]
        id_ss = [lax.reduce_max(v, axes=(0,)) for v in id_vecs]
        for t, id_s in zip(ts, id_ss):
            pltpu.async_copy(
                table_v.at[pl.ds(id_s, 1)],
                out_hbm.at[pl.ds(base + t, 1)],
                sem)
        return carry

    lax.fori_loop(0, b_per_w // UNROLL, tok_body, 0)

    def drain_body(t, carry):
        pltpu.make_async_copy(
            table_v.at[pl.ds(0, 1)], out_hbm.at[pl.ds(base, 1)], sem
        ).wait()
        return carry

    lax.fori_loop(0, b_per_w, drain_body, 0)


def kernel(token_type_ids, W_token_type):
    B, S = token_type_ids.shape
    D = W_token_type.shape[1]
    n_tok = B * S
    ids = token_type_ids.reshape(n_tok).astype(jnp.int32)
    mesh = plsc.VectorSubcoreMesh(
        core_axis_name="c", subcore_axis_name="s",
        num_cores=NC, num_subcores=NS,
    )
    out = pl.kernel(
        _sc_body,
        out_type=jax.ShapeDtypeStruct((n_tok, D), jnp.float32),
        mesh=mesh,
        compiler_params=pltpu.CompilerParams(needs_layout_passes=False),
        scratch_types=[
            pltpu.VMEM((n_tok // NW,), jnp.int32),
            pltpu.VMEM((2, D), jnp.float32),
            pltpu.SemaphoreType.DMA,
        ],
    )(ids, W_token_type)
    return out.reshape(B, S, D)


# speedup vs baseline: 1.0050x; 1.0050x over previous
import jax
import jax.numpy as jnp
from jax import lax
from jax.experimental import pallas as pl
from jax.experimental.pallas import tpu as pltpu
from jax.experimental.pallas import tpu_sc as plsc

NC = 2
NS = 16
NW = NC * NS
L = 16


def _sc_body(ids_hbm, table_hbm, out_hbm, idx_v, table_v, sem):
    wid = lax.axis_index("s") * NC + lax.axis_index("c")
    n_tok = ids_hbm.shape[0]
    b_per_w = n_tok // NW
    base = wid * b_per_w

    pltpu.sync_copy(table_hbm, table_v)
    pltpu.sync_copy(ids_hbm.at[pl.ds(base, b_per_w)], idx_v)

    def tok_body(t, carry):
        id_vec = plsc.load_gather(idx_v, [jnp.full((L,), t, jnp.int32)])
        id_s = lax.reduce_max(id_vec, axes=(0,))
        pltpu.async_copy(
            table_v.at[pl.ds(id_s, 1)],
            out_hbm.at[pl.ds(base + t, 1)],
            sem)
        return carry

    lax.fori_loop(0, b_per_w, tok_body, 0)

    def drain_body(t, carry):
        pltpu.make_async_copy(
            table_v.at[pl.ds(0, 1)], out_hbm.at[pl.ds(base, 1)], sem
        ).wait()
        return carry

    lax.fori_loop(0, b_per_w, drain_body, 0)


def kernel(token_type_ids, W_token_type):
    B, S = token_type_ids.shape
    D = W_token_type.shape[1]
    n_tok = B * S
    ids = token_type_ids.reshape(n_tok).astype(jnp.int32)
    mesh = plsc.VectorSubcoreMesh(
        core_axis_name="c", subcore_axis_name="s",
        num_cores=NC, num_subcores=NS,
    )
    out = pl.kernel(
        _sc_body,
        out_type=jax.ShapeDtypeStruct((n_tok, D), jnp.float32),
        mesh=mesh,
        compiler_params=pltpu.CompilerParams(needs_layout_passes=False),
        scratch_types=[
            pltpu.VMEM((n_tok // NW,), jnp.int32),
            pltpu.VMEM((2, D), jnp.float32),
            pltpu.SemaphoreType.DMA,
        ],
    )(ids, W_token_type)
    return out.reshape(B, S, D)
